# baseline (speedup 1.0000x reference)
import functools

import jax
import jax.numpy as jnp
from jax import lax
from jax.experimental import pallas as pl
from jax.experimental.pallas import tpu as pltpu
from jax.experimental.pallas import tpu_sc as plsc

_K = 32
_EPS = 1e-5
_MASKED = 1e30


def _topk_pq_kernel(n, rb, c, co, xb_ref, xt_ref, w1_ref, b1_ref,
                    sid_ref, p_ref, q_ref, pair_scr):
    i = pl.program_id(0)
    b = i // (n // rb)
    xb = xb_ref[...]
    xt = xt_ref[0]
    inner = lax.dot_general(xb, xt, (((1,), (0,)), ((), ())),
                            preferred_element_type=jnp.float32)
    sq_r = jnp.sum(xb * xb, axis=1, keepdims=True)
    sq_c = jnp.sum(xt * xt, axis=0, keepdims=True)
    pair_scr[...] = sq_r + sq_c - 2.0 * inner
    iota_f = lax.broadcasted_iota(jnp.int32, (rb, n), 1).astype(jnp.float32)
    cols = []
    sel = None
    for _ in range(_K):
        if sel is None:
            v = pair_scr[...]
        else:
            v = jnp.where(iota_f == sel, _MASKED, pair_scr[...])
            pair_scr[...] = v
        m = jnp.min(v, axis=1, keepdims=True)
        sel = jnp.min(jnp.where(v <= m, iota_f, float(n)),
                      axis=1, keepdims=True)
        cols.append(sel)
    sid_ref[...] = (jnp.concatenate(cols, axis=1).astype(jnp.int32) + b * n)
    w1 = w1_ref[...]
    wa = w1[:c] - w1[c:]
    p_ref[...] = lax.dot_general(xb, wa, (((1,), (0,)), ((), ())),
                                 preferred_element_type=jnp.float32,
                                 precision=lax.Precision.HIGHEST) + b1_ref[...]
    q = lax.dot_general(xb, w1[c:], (((1,), (0,)), ((), ())),
                        preferred_element_type=jnp.float32,
                        precision=lax.Precision.HIGHEST)
    q_ref[...] = jnp.concatenate([q, jnp.zeros_like(q)], axis=1)


def _knn_pq(x, W1, b1, rb=512):
    B, N, C = x.shape
    CO = W1.shape[1]
    BN = B * N
    x2d = x.reshape(BN, C)
    xt = x.transpose(0, 2, 1)
    grid = (BN // rb,)
    kern = functools.partial(_topk_pq_kernel, N, rb, C, CO)
    return pl.pallas_call(
        kern,
        grid=grid,
        in_specs=[
            pl.BlockSpec((rb, C), lambda i: (i, 0)),
            pl.BlockSpec((1, C, N), lambda i, n=N, r=rb: (i // (n // r), 0, 0)),
            pl.BlockSpec((2 * C, CO), lambda i: (0, 0)),
            pl.BlockSpec((1, CO), lambda i: (0, 0)),
        ],
        out_specs=[
            pl.BlockSpec((rb, _K), lambda i: (i, 0)),
            pl.BlockSpec((rb, CO), lambda i: (i, 0)),
            pl.BlockSpec((rb, 2 * CO), lambda i: (i, 0)),
        ],
        out_shape=[
            jax.ShapeDtypeStruct((BN, _K), jnp.int32),
            jax.ShapeDtypeStruct((BN, CO), jnp.float32),
            jax.ShapeDtypeStruct((BN, 2 * CO), jnp.float32),
        ],
        scratch_shapes=[pltpu.VMEM((rb, N), jnp.float32)],
    )(x2d, xt, W1, b1.reshape(1, CO))


def _sc_gather_reduce(Q, sid_flat):
    BN, CO2 = Q.shape
    CO = CO2 // 2
    info = plsc.get_sparse_core_info()
    NC, NS, L = info.num_cores, info.num_subcores, info.num_lanes
    NW = NC * NS
    RW = BN // NW
    GT = 4
    NG = RW // GT
    NCV = CO // L
    mesh = plsc.VectorSubcoreMesh(core_axis_name="c", subcore_axis_name="s")

    @functools.partial(
        pl.kernel, mesh=mesh,
        out_type=[jax.ShapeDtypeStruct((BN * CO,), jnp.float32)] * 4,
        scratch_types=[
            pltpu.VMEM((RW * _K,), jnp.int32),
            pltpu.VMEM((GT * _K, CO2), jnp.float32),
            pltpu.VMEM((GT * _K, CO2), jnp.float32),
            pltpu.VMEM((RW * CO,), jnp.float32),
            pltpu.VMEM((RW * CO,), jnp.float32),
            pltpu.VMEM((RW * CO,), jnp.float32),
            pltpu.VMEM((RW * CO,), jnp.float32),
            pltpu.SemaphoreType.DMA,
            pltpu.SemaphoreType.DMA,
        ],
    )
    def sck(q_hbm, idx_hbm, omax, omin, os1, os2,
            idxw, rows0, rows1, bmax, bmin, bs1, bs2, sem0, sem1):
        wid = lax.axis_index("s") * NC + lax.axis_index("c")
        base = wid * RW
        pltpu.sync_copy(idx_hbm.at[pl.ds(base * _K, RW * _K)], idxw)

        def start(g, rows, sem):
            pltpu.async_copy(q_hbm.at[idxw.at[pl.ds(g * GT * _K, GT * _K)]],
                             rows, sem)

        def compute(g, rows):
            for t in range(GT):
                for cb in range(NCV):
                    sl = pl.ds(cb * L, L)
                    v0 = rows[t * _K, sl]
                    amax = v0
                    amin = v0
                    asum = v0
                    asq = v0 * v0
                    for k in range(1, _K):
                        v = rows[t * _K + k, sl]
                        amax = jnp.maximum(amax, v)
                        amin = jnp.minimum(amin, v)
                        asum = asum + v
                        asq = asq + v * v
                    dst = pl.ds((g * GT + t) * CO + cb * L, L)
                    bmax[dst] = amax
                    bmin[dst] = amin
                    bs1[dst] = asum
                    bs2[dst] = asq

        start(0, rows0, sem0)

        def body(i, carry):
            g0 = 2 * i
            start(g0 + 1, rows1, sem1)
            pltpu.make_async_copy(
                q_hbm.at[idxw.at[pl.ds(0, GT * _K)]], rows0, sem0).wait()
            compute(g0, rows0)

            @pl.when(g0 + 2 < NG)
            def _():
                start(g0 + 2, rows0, sem0)

            pltpu.make_async_copy(
                q_hbm.at[idxw.at[pl.ds(0, GT * _K)]], rows1, sem1).wait()
            compute(g0 + 1, rows1)
            return carry

        lax.fori_loop(0, NG // 2, body, 0)
        for buf, out in ((bmax, omax), (bmin, omin), (bs1, os1), (bs2, os2)):
            pltpu.sync_copy(buf, out.at[pl.ds(base * CO, RW * CO)])

    outs = sck(Q, sid_flat)
    return [o.reshape(BN, CO) for o in outs]


def _finish_kernel(bn, co, p_ref, mx_ref, mn_ref, s1_ref, s2_ref,
                   g1_ref, bt1_ref, g2_ref, bt2_ref, out_ref):
    P = p_ref[...]
    s1 = s1_ref[...]
    s2 = s2_ref[...]
    bnk = bn * _K
    sum1 = jnp.sum(_K * P + s1, axis=0, keepdims=True)
    sum2 = jnp.sum(_K * P * P + 2.0 * P * s1 + s2, axis=0, keepdims=True)
    mean = sum1 / bnk
    var = sum2 / bnk - mean * mean
    a = g1_ref[...] * lax.rsqrt(var + _EPS)
    sh = bt1_ref[...] - mean * a
    M = jnp.where(a >= 0, P + mx_ref[...], P + mn_ref[...])
    xo = jnp.maximum(a * M + sh, 0.0)
    m2 = jnp.sum(xo, axis=0, keepdims=True) / bn
    v2 = jnp.sum(xo * xo, axis=0, keepdims=True) / bn - m2 * m2
    a2 = g2_ref[...] * lax.rsqrt(v2 + _EPS)
    out_ref[...] = jnp.maximum(a2 * xo + (bt2_ref[...] - m2 * a2), 0.0)


def _finish(P, mx, mn, s1, s2, gamma1, beta1, gamma2, beta2):
    BN, CO = P.shape
    kern = functools.partial(_finish_kernel, BN, CO)
    return pl.pallas_call(
        kern,
        out_shape=jax.ShapeDtypeStruct((BN, CO), jnp.float32),
    )(P, mx, mn, s1, s2,
      gamma1.reshape(1, CO), beta1.reshape(1, CO),
      gamma2.reshape(1, CO), beta2.reshape(1, CO))


def kernel(x, W1, b1, gamma1, beta1, gamma2, beta2):
    B, N, C = x.shape
    CO = W1.shape[1]
    parts = [_knn_pq(x[b:b + 1], W1, b1) for b in range(B)]
    stats = [_sc_gather_reduce(Q, sid.reshape(-1)) for sid, _, Q in parts]
    P = jnp.concatenate([p for _, p, _ in parts], axis=0)
    mx, mn, s1, s2 = (jnp.concatenate(t, axis=0) for t in zip(*stats))
    xo = _finish(P, mx, mn, s1, s2, gamma1, beta1, gamma2, beta2)
    return xo.reshape(B, N, CO)


# speedup vs baseline: 1.1922x; 1.1922x over previous
import functools

import jax
import jax.numpy as jnp
from jax import lax
from jax.experimental import pallas as pl
from jax.experimental.pallas import tpu as pltpu
from jax.experimental.pallas import tpu_sc as plsc

_K = 32
_EPS = 1e-5
_MASKED = 1e30


def _topk_pq_kernel(n, rb, c, co, xb_ref, xt_ref, w1_ref, b1_ref,
                    sid_ref, p_ref, q_ref, pair_scr):
    i = pl.program_id(0)
    b = i // (n // rb)
    xb = xb_ref[...]
    xt = xt_ref[0]
    inner = lax.dot_general(xb, xt, (((1,), (0,)), ((), ())),
                            preferred_element_type=jnp.float32)
    sq_r = jnp.sum(xb * xb, axis=1, keepdims=True)
    sq_c = jnp.sum(xt * xt, axis=0, keepdims=True)
    pair_scr[...] = sq_r + sq_c - 2.0 * inner
    iota_f = lax.broadcasted_iota(jnp.int32, (rb, n), 1).astype(jnp.float32)
    cols = []
    sel = None
    for _ in range(_K):
        if sel is None:
            v = pair_scr[...]
        else:
            v = jnp.where(iota_f == sel, _MASKED, pair_scr[...])
            pair_scr[...] = v
        m = jnp.min(v, axis=1, keepdims=True)
        sel = jnp.min(jnp.where(v <= m, iota_f, float(n)),
                      axis=1, keepdims=True)
        cols.append(sel)
    sid_ref[...] = (jnp.concatenate(cols, axis=1).astype(jnp.int32) + b * n)
    w1 = w1_ref[...]
    wa = w1[:c] - w1[c:]
    p_ref[...] = lax.dot_general(xb, wa, (((1,), (0,)), ((), ())),
                                 preferred_element_type=jnp.float32,
                                 precision=lax.Precision.HIGHEST) + b1_ref[...]
    q = lax.dot_general(xb, w1[c:], (((1,), (0,)), ((), ())),
                        preferred_element_type=jnp.float32,
                        precision=lax.Precision.HIGHEST)
    q_ref[...] = jnp.concatenate([q, jnp.zeros_like(q)], axis=1)


def _knn_pq(x, W1, b1, rb=128):
    B, N, C = x.shape
    CO = W1.shape[1]
    BN = B * N
    x2d = x.reshape(BN, C)
    xt = x.transpose(0, 2, 1)
    grid = (BN // rb,)
    kern = functools.partial(_topk_pq_kernel, N, rb, C, CO)
    return pl.pallas_call(
        kern,
        grid=grid,
        in_specs=[
            pl.BlockSpec((rb, C), lambda i: (i, 0)),
            pl.BlockSpec((1, C, N), lambda i, n=N, r=rb: (i // (n // r), 0, 0)),
            pl.BlockSpec((2 * C, CO), lambda i: (0, 0)),
            pl.BlockSpec((1, CO), lambda i: (0, 0)),
        ],
        out_specs=[
            pl.BlockSpec((rb, _K), lambda i: (i, 0)),
            pl.BlockSpec((rb, CO), lambda i: (i, 0)),
            pl.BlockSpec((rb, 2 * CO), lambda i: (i, 0)),
        ],
        out_shape=[
            jax.ShapeDtypeStruct((BN, _K), jnp.int32),
            jax.ShapeDtypeStruct((BN, CO), jnp.float32),
            jax.ShapeDtypeStruct((BN, 2 * CO), jnp.float32),
        ],
        scratch_shapes=[pltpu.VMEM((rb, N), jnp.float32)],
    )(x2d, xt, W1, b1.reshape(1, CO))


def _sc_gather_reduce(Q, sid_flat):
    BN, CO2 = Q.shape
    CO = CO2 // 2
    info = plsc.get_sparse_core_info()
    NC, NS, L = info.num_cores, info.num_subcores, info.num_lanes
    NW = NC * NS
    RW = BN // NW
    GT = 4
    NG = RW // GT
    NCV = CO // L
    mesh = plsc.VectorSubcoreMesh(core_axis_name="c", subcore_axis_name="s")

    @functools.partial(
        pl.kernel, mesh=mesh,
        out_type=[jax.ShapeDtypeStruct((BN * CO,), jnp.float32)] * 4,
        scratch_types=[
            pltpu.VMEM((RW * _K,), jnp.int32),
            pltpu.VMEM((GT * _K, CO2), jnp.float32),
            pltpu.VMEM((GT * _K, CO2), jnp.float32),
            pltpu.VMEM((RW * CO,), jnp.float32),
            pltpu.VMEM((RW * CO,), jnp.float32),
            pltpu.VMEM((RW * CO,), jnp.float32),
            pltpu.VMEM((RW * CO,), jnp.float32),
            pltpu.SemaphoreType.DMA,
            pltpu.SemaphoreType.DMA,
        ],
    )
    def sck(q_hbm, idx_hbm, omax, omin, os1, os2,
            idxw, rows0, rows1, bmax, bmin, bs1, bs2, sem0, sem1):
        wid = lax.axis_index("s") * NC + lax.axis_index("c")
        base = wid * RW
        pltpu.sync_copy(idx_hbm.at[pl.ds(base * _K, RW * _K)], idxw)

        def start(g, rows, sem):
            pltpu.async_copy(q_hbm.at[idxw.at[pl.ds(g * GT * _K, GT * _K)]],
                             rows, sem)

        def compute(g, rows):
            for t in range(GT):
                for cb in range(NCV):
                    sl = pl.ds(cb * L, L)
                    v0 = rows[t * _K, sl]
                    amax = v0
                    amin = v0
                    asum = v0
                    asq = v0 * v0
                    for k in range(1, _K):
                        v = rows[t * _K + k, sl]
                        amax = jnp.maximum(amax, v)
                        amin = jnp.minimum(amin, v)
                        asum = asum + v
                        asq = asq + v * v
                    dst = pl.ds((g * GT + t) * CO + cb * L, L)
                    bmax[dst] = amax
                    bmin[dst] = amin
                    bs1[dst] = asum
                    bs2[dst] = asq

        start(0, rows0, sem0)

        def body(i, carry):
            g0 = 2 * i
            start(g0 + 1, rows1, sem1)
            pltpu.make_async_copy(
                q_hbm.at[idxw.at[pl.ds(0, GT * _K)]], rows0, sem0).wait()
            compute(g0, rows0)

            @pl.when(g0 + 2 < NG)
            def _():
                start(g0 + 2, rows0, sem0)

            pltpu.make_async_copy(
                q_hbm.at[idxw.at[pl.ds(0, GT * _K)]], rows1, sem1).wait()
            compute(g0 + 1, rows1)
            return carry

        lax.fori_loop(0, NG // 2, body, 0)
        for buf, out in ((bmax, omax), (bmin, omin), (bs1, os1), (bs2, os2)):
            pltpu.sync_copy(buf, out.at[pl.ds(base * CO, RW * CO)])

    outs = sck(Q, sid_flat)
    return [o.reshape(BN, CO) for o in outs]


def _finish_kernel(bn, co, p_ref, mx_ref, mn_ref, s1_ref, s2_ref,
                   g1_ref, bt1_ref, g2_ref, bt2_ref, out_ref):
    P = p_ref[...]
    s1 = s1_ref[...]
    s2 = s2_ref[...]
    bnk = bn * _K
    sum1 = jnp.sum(_K * P + s1, axis=0, keepdims=True)
    sum2 = jnp.sum(_K * P * P + 2.0 * P * s1 + s2, axis=0, keepdims=True)
    mean = sum1 / bnk
    var = sum2 / bnk - mean * mean
    a = g1_ref[...] * lax.rsqrt(var + _EPS)
    sh = bt1_ref[...] - mean * a
    M = jnp.where(a >= 0, P + mx_ref[...], P + mn_ref[...])
    xo = jnp.maximum(a * M + sh, 0.0)
    m2 = jnp.sum(xo, axis=0, keepdims=True) / bn
    v2 = jnp.sum(xo * xo, axis=0, keepdims=True) / bn - m2 * m2
    a2 = g2_ref[...] * lax.rsqrt(v2 + _EPS)
    out_ref[...] = jnp.maximum(a2 * xo + (bt2_ref[...] - m2 * a2), 0.0)


def _finish(P, mx, mn, s1, s2, gamma1, beta1, gamma2, beta2):
    BN, CO = P.shape
    kern = functools.partial(_finish_kernel, BN, CO)
    return pl.pallas_call(
        kern,
        out_shape=jax.ShapeDtypeStruct((BN, CO), jnp.float32),
    )(P, mx, mn, s1, s2,
      gamma1.reshape(1, CO), beta1.reshape(1, CO),
      gamma2.reshape(1, CO), beta2.reshape(1, CO))


def kernel(x, W1, b1, gamma1, beta1, gamma2, beta2):
    B, N, C = x.shape
    CO = W1.shape[1]
    parts = [_knn_pq(x[b:b + 1], W1, b1) for b in range(B)]
    stats = [_sc_gather_reduce(Q, sid.reshape(-1)) for sid, _, Q in parts]
    P = jnp.concatenate([p for _, p, _ in parts], axis=0)
    mx, mn, s1, s2 = (jnp.concatenate(t, axis=0) for t in zip(*stats))
    xo = _finish(P, mx, mn, s1, s2, gamma1, beta1, gamma2, beta2)
    return xo.reshape(B, N, CO)
